# baseline (speedup 1.0000x reference)
import functools

import jax
import jax.numpy as jnp
from jax import lax
from jax.experimental import pallas as pl
from jax.experimental.pallas import tpu as pltpu
from jax.experimental.pallas import tpu_sc as plsc

F32 = jnp.float32
I32 = jnp.int32

NC = 2
NS = 16
NW = NC * NS
CH = 128


CHA = 96


def _make_passA(l, ne_pad, n_out, h_heads, hc):
    pw = ne_pad // NW
    nchunks = pw // CHA
    mesh = plsc.VectorSubcoreMesh(core_axis_name="c", subcore_axis_name="s")

    def body(src_hbm, dst_hbm, xl_hbm, xr_hbm, em_hbm, att_hbm,
             ex_hbm,
             srcv0, dstv0, xlr0, xrr0, emr0, exb0,
             srcv1, dstv1, xlr1, xrr1, emr1, exb1,
             attv, semA, semB):
        c = lax.axis_index("c")
        s = lax.axis_index("s")
        wid = s * NC + c
        pltpu.sync_copy(att_hbm, attv)
        iot = lax.iota(I32, 16)
        lastm = iot == (hc - 1)
        hcols = [jnp.full((16,), h, I32) for h in range(h_heads)]
        bufs = [(srcv0, dstv0, xlr0, xrr0, emr0, exb0, semA),
                (srcv1, dstv1, xlr1, xrr1, emr1, exb1, semB)]

        def issue(ci, b):
            srcv, dstv, xlr, xrr, emr, _, sem = bufs[b]
            base = wid * pw + ci * CHA
            pltpu.sync_copy(src_hbm.at[pl.ds(base, CHA)], srcv)
            pltpu.sync_copy(dst_hbm.at[pl.ds(base, CHA)], dstv)
            pltpu.async_copy(xl_hbm.at[srcv], xlr, sem)
            pltpu.async_copy(xr_hbm.at[dstv], xrr, sem)
            pltpu.async_copy(em_hbm.at[l].at[pl.ds(base, CHA)], emr, sem)

        def drain(b):
            srcv, dstv, xlr, xrr, emr, _, sem = bufs[b]
            pltpu.make_async_copy(xl_hbm.at[srcv], xlr, sem).wait()
            pltpu.make_async_copy(xr_hbm.at[dstv], xrr, sem).wait()
            pltpu.make_async_copy(em_hbm.at[l].at[pl.ds(0, CHA)], emr, sem).wait()

        issue(0, 0)

        def two(i2, carry):
            for b in (0, 1):
                ci = i2 * 2 + b
                nb = 1 - b
                srcv, dstv, xlr, xrr, emr, exb, sem = bufs[b]

                @pl.when(ci + 1 < nchunks)
                def _():
                    issue(ci + 1, nb)

                drain(b)

                @plsc.parallel_loop(0, CHA, 1, unroll=4)
                def edge(e):
                    erow = jnp.full((16,), e, I32)
                    for h in range(h_heads):
                        sl = pl.ds(h * hc, hc)
                        m = xlr[e, sl] + xrr[e, sl] + emr[e, sl]
                        m = jnp.maximum(m, m * 0.2)
                        sv = m * attv[h, :]
                        cs = plsc.cumsum(sv)
                        ev = jnp.exp(cs)
                        plsc.store_scatter(exb, [erow, hcols[h]], ev, mask=lastm)

                base = wid * pw + ci * CHA
                pltpu.sync_copy(exb, ex_hbm.at[pl.ds(base, CHA)])
            return carry

        lax.fori_loop(0, nchunks // 2, two, 0)

    vm = pltpu.VMEM
    return pl.kernel(
        body,
        out_type=jax.ShapeDtypeStruct((ne_pad, h_heads), F32),
        mesh=mesh,
        compiler_params=pltpu.CompilerParams(needs_layout_passes=False),
        scratch_types=[
            vm((CHA,), I32), vm((CHA,), I32),
            vm((CHA, h_heads * hc), F32), vm((CHA, h_heads * hc), F32),
            vm((CHA, h_heads * hc), F32), vm((CHA, h_heads), F32),
            vm((CHA,), I32), vm((CHA,), I32),
            vm((CHA, h_heads * hc), F32), vm((CHA, h_heads * hc), F32),
            vm((CHA, h_heads * hc), F32), vm((CHA, h_heads), F32),
            vm((h_heads, hc), F32),
            pltpu.SemaphoreType.DMA,
            pltpu.SemaphoreType.DMA,
        ],
    )


CHD = 288


def _make_passD(ne_pad, n_out, h_heads):
    pw = ne_pad // NW
    nchunks = pw // CHD
    mesh = plsc.VectorSubcoreMesh(core_axis_name="c", subcore_axis_name="s")

    def body(dst_hbm, ex_hbm, den_hbm, dstv, exr, dtile):
        c = lax.axis_index("c")
        s = lax.axis_index("s")
        wid = s * NC + c
        iot = lax.iota(I32, 16)
        zeros16 = jnp.zeros((16,), F32)

        @plsc.parallel_loop(0, n_out * h_heads // 16, 1, unroll=4)
        def zden(j):
            dtile[pl.ds(j * 16, 16)] = zeros16

        def chunk(i, carry):
            base = wid * pw + i * CHD
            pltpu.sync_copy(dst_hbm.at[pl.ds(base, CHD)], dstv)
            pltpu.sync_copy(ex_hbm.at[pl.ds(base, CHD)], exr)

            @plsc.parallel_loop(0, CHD * h_heads // 16, 1, unroll=4)
            def grp(g):
                fl = g * 16 + iot
                e16 = lax.shift_right_logical(fl, 3)
                h16 = fl & 7
                d16 = plsc.load_gather(dstv, [e16])
                exv = plsc.load_gather(exr, [e16, h16])
                plsc.addupdate_scatter(dtile, [d16 * h_heads + h16], exv)

            return carry

        lax.fori_loop(0, nchunks, chunk, 0)
        pltpu.sync_copy(dtile, den_hbm.at[wid])

    return pl.kernel(
        body,
        out_type=jax.ShapeDtypeStruct((NW, n_out * h_heads), F32),
        mesh=mesh,
        compiler_params=pltpu.CompilerParams(needs_layout_passes=False),
        scratch_types=[
            pltpu.VMEM((CHD,), I32),
            pltpu.VMEM((CHD, h_heads), F32),
            pltpu.VMEM((n_out * h_heads,), F32),
        ],
    )


CHB = 64


def _make_passB2(ne_pad, n_out, h_heads, hc):
    pw = ne_pad // NW
    nchunks = pw // CHB
    rpt = n_out // NS
    mesh = plsc.VectorSubcoreMesh(core_axis_name="c", subcore_axis_name="s")

    def body(src_hbm, dst_hbm, xl_hbm, ex_hbm, z128_hbm,
             out_hbm,
             srcv0, dstv0, dstS0, xlr0, exr0, wr0,
             srcv1, dstv1, dstS1, xlr1, exr1, wr1,
             out_sh, semG0, semG1, semS0, semS1, semZ):
        c = lax.axis_index("c")
        s = lax.axis_index("s")
        wid = s * NC + c
        r0 = s * rpt

        def zfire(j, zcarry):
            sl = pl.ds(r0 + 8 * j, 8)
            pltpu.async_copy(z128_hbm.at[sl], out_sh.at[sl], semZ)
            return zcarry

        def zdrain(j, zcarry):
            sl0 = pl.ds(r0, 8)
            pltpu.make_async_copy(z128_hbm.at[sl0], out_sh.at[sl0], semZ).wait()
            return zcarry

        lax.fori_loop(0, rpt // 8, zfire, 0)
        lax.fori_loop(0, rpt // 8, zdrain, 0)
        plsc.subcore_barrier()
        iot = lax.iota(I32, 16)
        hsel = iot & 7
        bufs = [(srcv0, dstv0, dstS0, xlr0, exr0, wr0, semG0, semS0),
                (srcv1, dstv1, dstS1, xlr1, exr1, wr1, semG1, semS1)]

        def issue(ci, b):
            srcv, dstv, _, xlr, exr, _, semG, _ = bufs[b]
            base = wid * pw + ci * CHB
            pltpu.sync_copy(src_hbm.at[pl.ds(base, CHB)], srcv)
            pltpu.sync_copy(dst_hbm.at[pl.ds(base, CHB)], dstv)
            pltpu.async_copy(xl_hbm.at[srcv], xlr, semG)
            pltpu.async_copy(ex_hbm.at[pl.ds(base, CHB)], exr, semG)

        def drain_gather(b):
            srcv, _, _, xlr, exr, _, semG, _ = bufs[b]
            pltpu.make_async_copy(xl_hbm.at[srcv], xlr, semG).wait()
            pltpu.make_async_copy(ex_hbm.at[pl.ds(0, CHB)], exr, semG).wait()

        def drain_scatter(b):
            _, _, dstS, _, _, wr, _, semS = bufs[b]
            pltpu.make_async_copy(wr, out_sh.at[dstS], semS).wait()

        issue(0, 0)

        def two(i2, carry):
            for b in (0, 1):
                ci = i2 * 2 + b
                nb = 1 - b
                srcv, dstv, dstS, xlr, exr, wr, semG, semS = bufs[b]

                @pl.when(ci + 1 < nchunks)
                def _():
                    issue(ci + 1, nb)

                drain_gather(b)

                @pl.when(ci >= 2)
                def _():
                    drain_scatter(b)

                @plsc.parallel_loop(0, CHB, 1, unroll=4)
                def edge(e):
                    erow = jnp.full((16,), e, I32)
                    a16 = plsc.load_gather(exr, [erow, hsel])
                    for h in range(h_heads):
                        sl = pl.ds(h * hc, hc)
                        wr[e, sl] = xlr[e, sl] * a16[h]

                for k in range(CHB // 16):
                    ksl = pl.ds(k * 16, 16)
                    dstS[ksl] = dstv[ksl]
                pltpu.async_copy(wr, out_sh.at[dstS], semS, add=True)
            return carry

        lax.fori_loop(0, nchunks // 2, two, 0)
        drain_scatter(0)
        drain_scatter(1)
        plsc.subcore_barrier()

        def ofire(j, ocarry):
            sl = pl.ds(r0 + 8 * j, 8)
            pltpu.async_copy(out_sh.at[sl], out_hbm.at[c].at[sl], semZ)
            return ocarry

        def odrain(j, ocarry):
            sl0 = pl.ds(r0, 8)
            pltpu.make_async_copy(
                out_sh.at[sl0], out_hbm.at[c].at[sl0], semZ).wait()
            return ocarry

        lax.fori_loop(0, rpt // 8, ofire, 0)
        lax.fori_loop(0, rpt // 8, odrain, 0)

    vm = pltpu.VMEM
    return pl.kernel(
        body,
        out_type=jax.ShapeDtypeStruct((NC, n_out, h_heads * hc), F32),
        mesh=mesh,
        compiler_params=pltpu.CompilerParams(needs_layout_passes=False),
        scratch_types=[
            vm((CHB,), I32), vm((CHB,), I32), vm((CHB,), I32),
            vm((CHB, h_heads * hc), F32), vm((CHB, h_heads), F32),
            vm((CHB, h_heads * hc), F32),
            vm((CHB,), I32), vm((CHB,), I32), vm((CHB,), I32),
            vm((CHB, h_heads * hc), F32), vm((CHB, h_heads), F32),
            vm((CHB, h_heads * hc), F32),
            pltpu.VMEM_SHARED((n_out, h_heads * hc), F32),
            pltpu.SemaphoreType.DMA, pltpu.SemaphoreType.DMA,
            pltpu.SemaphoreType.DMA, pltpu.SemaphoreType.DMA,
            pltpu.SemaphoreType.DMA,
        ],
    )


def _tc_init_body(x_ref, wl_ref, bl_ref, wr_ref, br_ref, xl_ref, xr_ref):
    xb = x_ref[...]
    xl_ref[...] = jnp.dot(xb, wl_ref[...], preferred_element_type=F32) + bl_ref[...]
    xr_ref[...] = jnp.dot(xb, wr_ref[...], preferred_element_type=F32) + br_ref[...]


def _tc_em_body(ea_ref, we_ref, em_ref):
    em_ref[0] = jnp.dot(ea_ref[...], we_ref[0], preferred_element_type=F32)


def _den_widen(den_ref, nb, nheads, hc):
    dnb = jnp.sum(den_ref[...], axis=0) + 1e-16
    return jnp.concatenate(
        [jnp.broadcast_to(dnb[:, h:h + 1], (nb, hc)) for h in range(nheads)],
        axis=1)


def _tc_epi_body(nb, nheads, hc, p_ref, den_ref, cb_ref, lg_ref, lb_ref,
                 h_ref, wl_ref, bl_ref, wr_ref, br_ref, h_out, xl_out, xr_out):
    o = (p_ref[0] + p_ref[1]) / _den_widen(den_ref, nb, nheads, hc) + cb_ref[...]
    o = jnp.where(o > 0, o, jnp.exp(o) - 1.0)
    mu = jnp.mean(o, axis=-1, keepdims=True)
    var = jnp.mean((o - mu) * (o - mu), axis=-1, keepdims=True)
    o = (o - mu) / jnp.sqrt(var + 1e-5) * lg_ref[...] + lb_ref[...]
    hn = o + h_ref[...]
    h_out[...] = hn
    xl_out[...] = jnp.dot(hn, wl_ref[...], preferred_element_type=F32) + bl_ref[...]
    xr_out[...] = jnp.dot(hn, wr_ref[...], preferred_element_type=F32) + br_ref[...]


def _tc_final_body(nb, nheads, hc, p_ref, den_ref, cb_ref, lg_ref, lb_ref,
                   wp_ref, bp_ref, y_ref):
    o = (p_ref[0] + p_ref[1]) / _den_widen(den_ref, nb, nheads, hc)
    mh = o[:, 0:hc]
    for h in range(1, nheads):
        mh = mh + o[:, h * hc:(h + 1) * hc]
    mh = mh * (1.0 / nheads) + cb_ref[...]
    mh = jnp.where(mh > 0, mh, jnp.exp(mh) - 1.0)
    mu = jnp.mean(mh, axis=-1, keepdims=True)
    var = jnp.mean((mh - mu) * (mh - mu), axis=-1, keepdims=True)
    mh = (mh - mu) / jnp.sqrt(var + 1e-5) * lg_ref[...] + lb_ref[...]
    y_ref[...] = jnp.dot(mh, wp_ref[...], preferred_element_type=F32) + bp_ref[...]


def kernel(x, edge_index, edge_attr,
           Wl0, bl0, Wr0, br0, att0, We0, cb0, lg0, lb0,
           Wl1, bl1, Wr1, br1, att1, We1, cb1, lg1, lb1,
           Wl2, bl2, Wr2, br2, att2, We2, cb2, lg2, lb2,
           Wp, bp):
    n, d = x.shape
    e = edge_index.shape[1]
    hh, hc = att0.shape
    dhid = hh * hc
    ne = e + n
    ne_pad = ((ne + NW * CH - 1) // (NW * CH)) * (NW * CH)
    n_out = ((n + 1 + NS * 8 - 1) // (NS * 8)) * (NS * 8)
    nb = 1000
    ngrid = n // nb
    eb = 4096
    egrid = ne_pad // eb

    loop = jnp.arange(n, dtype=edge_index.dtype)
    pad = ne_pad - ne
    src = jnp.concatenate([edge_index[0], loop,
                           jnp.zeros((pad,), edge_index.dtype)])
    dst = jnp.concatenate([edge_index[1], loop,
                           jnp.full((pad,), n, edge_index.dtype)])
    fill = jnp.mean(edge_attr, axis=0, keepdims=True)
    ea = jnp.concatenate([edge_attr, jnp.tile(fill, (n, 1)),
                          jnp.zeros((pad, edge_attr.shape[1]), F32)])
    we_all = jnp.stack([We0, We1, We2])
    z128 = jnp.zeros((n_out, dhid), F32)

    xl, xr = pl.pallas_call(
        _tc_init_body,
        grid=(ngrid,),
        in_specs=[
            pl.BlockSpec((nb, d), lambda i: (i, 0)),
            pl.BlockSpec((d, dhid), lambda i: (0, 0)),
            pl.BlockSpec((1, dhid), lambda i: (0, 0)),
            pl.BlockSpec((d, dhid), lambda i: (0, 0)),
            pl.BlockSpec((1, dhid), lambda i: (0, 0)),
        ],
        out_specs=[
            pl.BlockSpec((nb, dhid), lambda i: (i, 0)),
            pl.BlockSpec((nb, dhid), lambda i: (i, 0)),
        ],
        out_shape=[
            jax.ShapeDtypeStruct((n, dhid), F32),
            jax.ShapeDtypeStruct((n, dhid), F32),
        ],
    )(x, Wl0, bl0.reshape(1, -1), Wr0, br0.reshape(1, -1))

    em = pl.pallas_call(
        _tc_em_body,
        grid=(3, egrid),
        in_specs=[
            pl.BlockSpec((eb, ea.shape[1]), lambda l, j: (j, 0)),
            pl.BlockSpec((1, ea.shape[1], dhid), lambda l, j: (l, 0, 0)),
        ],
        out_specs=pl.BlockSpec((1, eb, dhid), lambda l, j: (l, j, 0)),
        out_shape=jax.ShapeDtypeStruct((3, ne_pad, dhid), F32),
    )(ea, we_all)

    layer_params = [
        (cb0, lg0, lb0, Wl1, bl1, Wr1, br1, att0),
        (cb1, lg1, lb1, Wl2, bl2, Wr2, br2, att1),
        (cb2, lg2, lb2, None, None, None, None, att2),
    ]

    h = x
    y = None
    for l in range(3):
        cb, lg, lb, wl_n, bl_n, wr_n, br_n, att = layer_params[l]
        passA = _make_passA(l, ne_pad, n_out, hh, hc)
        ex = passA(src, dst, xl, xr, em, att)
        passD = _make_passD(ne_pad, n_out, hh)
        den = passD(dst, ex)
        dent = den.reshape(NW, n_out, hh)
        passB2 = _make_passB2(ne_pad, n_out, hh, hc)
        outp = passB2(src, dst, xl, ex, z128)

        if l < 2:
            h, xl, xr = pl.pallas_call(
                functools.partial(_tc_epi_body, nb, hh, hc),
                grid=(ngrid,),
                in_specs=[
                    pl.BlockSpec((NC, nb, dhid), lambda i: (0, i, 0)),
                    pl.BlockSpec((NW, nb, hh), lambda i: (0, i, 0)),
                    pl.BlockSpec((1, dhid), lambda i: (0, 0)),
                    pl.BlockSpec((1, dhid), lambda i: (0, 0)),
                    pl.BlockSpec((1, dhid), lambda i: (0, 0)),
                    pl.BlockSpec((nb, dhid), lambda i: (i, 0)),
                    pl.BlockSpec((d, dhid), lambda i: (0, 0)),
                    pl.BlockSpec((1, dhid), lambda i: (0, 0)),
                    pl.BlockSpec((d, dhid), lambda i: (0, 0)),
                    pl.BlockSpec((1, dhid), lambda i: (0, 0)),
                ],
                out_specs=[
                    pl.BlockSpec((nb, dhid), lambda i: (i, 0)),
                    pl.BlockSpec((nb, dhid), lambda i: (i, 0)),
                    pl.BlockSpec((nb, dhid), lambda i: (i, 0)),
                ],
                out_shape=[
                    jax.ShapeDtypeStruct((n, dhid), F32),
                    jax.ShapeDtypeStruct((n, dhid), F32),
                    jax.ShapeDtypeStruct((n, dhid), F32),
                ],
            )(outp, dent, cb.reshape(1, -1), lg.reshape(1, -1),
              lb.reshape(1, -1), h, wl_n, bl_n.reshape(1, -1), wr_n,
              br_n.reshape(1, -1))
        else:
            y = pl.pallas_call(
                functools.partial(_tc_final_body, nb, hh, hc),
                grid=(ngrid,),
                in_specs=[
                    pl.BlockSpec((NC, nb, dhid), lambda i: (0, i, 0)),
                    pl.BlockSpec((NW, nb, hh), lambda i: (0, i, 0)),
                    pl.BlockSpec((1, hc), lambda i: (0, 0)),
                    pl.BlockSpec((1, hc), lambda i: (0, 0)),
                    pl.BlockSpec((1, hc), lambda i: (0, 0)),
                    pl.BlockSpec((hc, d), lambda i: (0, 0)),
                    pl.BlockSpec((1, d), lambda i: (0, 0)),
                ],
                out_specs=pl.BlockSpec((nb, d), lambda i: (i, 0)),
                out_shape=jax.ShapeDtypeStruct((n, d), F32),
            )(outp, dent, cb.reshape(1, -1), lg.reshape(1, -1),
              lb.reshape(1, -1), Wp, bp.reshape(1, -1))
    return y


# speedup vs baseline: 1.0601x; 1.0601x over previous
import functools

import jax
import jax.numpy as jnp
from jax import lax
from jax.experimental import pallas as pl
from jax.experimental.pallas import tpu as pltpu
from jax.experimental.pallas import tpu_sc as plsc

F32 = jnp.float32
I32 = jnp.int32

NC = 2
NS = 16
NW = NC * NS
CH = 128


CHA = 96


def _make_passA(l, ne_pad, n_out, h_heads, hc):
    pw = ne_pad // NW
    nchunks = pw // CHA
    mesh = plsc.VectorSubcoreMesh(core_axis_name="c", subcore_axis_name="s")

    def body(src_hbm, dst_hbm, xl_hbm, xr_hbm, em_hbm, att_hbm,
             ex_hbm,
             srcv0, dstv0, xlr0, xrr0, emr0, exb0,
             srcv1, dstv1, xlr1, xrr1, emr1, exb1,
             attv, semA, semB):
        c = lax.axis_index("c")
        s = lax.axis_index("s")
        wid = s * NC + c
        pltpu.sync_copy(att_hbm, attv)
        iot = lax.iota(I32, 16)
        lastm = iot == (hc - 1)
        hcols = [jnp.full((16,), h, I32) for h in range(h_heads)]
        bufs = [(srcv0, dstv0, xlr0, xrr0, emr0, exb0, semA),
                (srcv1, dstv1, xlr1, xrr1, emr1, exb1, semB)]

        def issue(ci, b):
            srcv, dstv, xlr, xrr, emr, _, sem = bufs[b]
            base = wid * pw + ci * CHA
            pltpu.sync_copy(src_hbm.at[pl.ds(base, CHA)], srcv)
            pltpu.sync_copy(dst_hbm.at[pl.ds(base, CHA)], dstv)
            pltpu.async_copy(xl_hbm.at[srcv], xlr, sem)
            pltpu.async_copy(xr_hbm.at[dstv], xrr, sem)
            pltpu.async_copy(em_hbm.at[l].at[pl.ds(base, CHA)], emr, sem)

        def drain(b):
            srcv, dstv, xlr, xrr, emr, _, sem = bufs[b]
            pltpu.make_async_copy(xl_hbm.at[srcv], xlr, sem).wait()
            pltpu.make_async_copy(xr_hbm.at[dstv], xrr, sem).wait()
            pltpu.make_async_copy(em_hbm.at[l].at[pl.ds(0, CHA)], emr, sem).wait()

        issue(0, 0)

        def two(i2, carry):
            for b in (0, 1):
                ci = i2 * 2 + b
                nb = 1 - b
                srcv, dstv, xlr, xrr, emr, exb, sem = bufs[b]

                @pl.when(ci + 1 < nchunks)
                def _():
                    issue(ci + 1, nb)

                drain(b)

                @plsc.parallel_loop(0, CHA, 1, unroll=4)
                def edge(e):
                    erow = jnp.full((16,), e, I32)
                    for h in range(h_heads):
                        sl = pl.ds(h * hc, hc)
                        m = xlr[e, sl] + xrr[e, sl] + emr[e, sl]
                        m = jnp.maximum(m, m * 0.2)
                        sv = m * attv[h, :]
                        cs = plsc.cumsum(sv)
                        ev = jnp.exp(cs)
                        plsc.store_scatter(exb, [erow, hcols[h]], ev, mask=lastm)

                base = wid * pw + ci * CHA
                pltpu.sync_copy(exb, ex_hbm.at[pl.ds(base, CHA)])
            return carry

        lax.fori_loop(0, nchunks // 2, two, 0)

    vm = pltpu.VMEM
    return pl.kernel(
        body,
        out_type=jax.ShapeDtypeStruct((ne_pad, h_heads), F32),
        mesh=mesh,
        compiler_params=pltpu.CompilerParams(needs_layout_passes=False),
        scratch_types=[
            vm((CHA,), I32), vm((CHA,), I32),
            vm((CHA, h_heads * hc), F32), vm((CHA, h_heads * hc), F32),
            vm((CHA, h_heads * hc), F32), vm((CHA, h_heads), F32),
            vm((CHA,), I32), vm((CHA,), I32),
            vm((CHA, h_heads * hc), F32), vm((CHA, h_heads * hc), F32),
            vm((CHA, h_heads * hc), F32), vm((CHA, h_heads), F32),
            vm((h_heads, hc), F32),
            pltpu.SemaphoreType.DMA,
            pltpu.SemaphoreType.DMA,
        ],
    )


CHD = 288


def _make_passD(ne_pad, n_out, h_heads):
    pw = ne_pad // NW
    nchunks = pw // CHD
    mesh = plsc.VectorSubcoreMesh(core_axis_name="c", subcore_axis_name="s")

    def body(dst_hbm, ex_hbm, den_hbm, dstv, exr, dtile):
        c = lax.axis_index("c")
        s = lax.axis_index("s")
        wid = s * NC + c
        iot = lax.iota(I32, 16)
        zeros16 = jnp.zeros((16,), F32)

        @plsc.parallel_loop(0, n_out * h_heads // 16, 1, unroll=4)
        def zden(j):
            dtile[pl.ds(j * 16, 16)] = zeros16

        def chunk(i, carry):
            base = wid * pw + i * CHD
            pltpu.sync_copy(dst_hbm.at[pl.ds(base, CHD)], dstv)
            pltpu.sync_copy(ex_hbm.at[pl.ds(base, CHD)], exr)

            @plsc.parallel_loop(0, CHD * h_heads // 16, 1, unroll=4)
            def grp(g):
                fl = g * 16 + iot
                e16 = lax.shift_right_logical(fl, 3)
                h16 = fl & 7
                d16 = plsc.load_gather(dstv, [e16])
                exv = plsc.load_gather(exr, [e16, h16])
                plsc.addupdate_scatter(dtile, [d16 * h_heads + h16], exv)

            return carry

        lax.fori_loop(0, nchunks, chunk, 0)
        pltpu.sync_copy(dtile, den_hbm.at[wid])

    return pl.kernel(
        body,
        out_type=jax.ShapeDtypeStruct((NW, n_out * h_heads), F32),
        mesh=mesh,
        compiler_params=pltpu.CompilerParams(needs_layout_passes=False),
        scratch_types=[
            pltpu.VMEM((CHD,), I32),
            pltpu.VMEM((CHD, h_heads), F32),
            pltpu.VMEM((n_out * h_heads,), F32),
        ],
    )


CHB = 64


def _make_passB2(ne_pad, n_out, h_heads, hc):
    pw = ne_pad // NW
    nchunks = pw // CHB
    rpt = n_out // NS
    mesh = plsc.VectorSubcoreMesh(core_axis_name="c", subcore_axis_name="s")

    def body(src_hbm, dst_hbm, xl_hbm, ex_hbm, z128_hbm,
             out_hbm,
             srcv0, dstv0, dstS0, xlr0, exr0, wr0,
             srcv1, dstv1, dstS1, xlr1, exr1, wr1,
             out_sh, semG0, semG1, semS0, semS1, semZ):
        c = lax.axis_index("c")
        s = lax.axis_index("s")
        wid = s * NC + c
        r0 = s * rpt

        def zfire(j, zcarry):
            sl = pl.ds(r0 + 8 * j, 8)
            pltpu.async_copy(z128_hbm.at[sl], out_sh.at[sl], semZ)
            return zcarry

        def zdrain(j, zcarry):
            sl0 = pl.ds(r0, 8)
            pltpu.make_async_copy(z128_hbm.at[sl0], out_sh.at[sl0], semZ).wait()
            return zcarry

        lax.fori_loop(0, rpt // 8, zfire, 0)
        lax.fori_loop(0, rpt // 8, zdrain, 0)
        plsc.subcore_barrier()
        iot = lax.iota(I32, 16)
        hsel = iot & 7
        bufs = [(srcv0, dstv0, dstS0, xlr0, exr0, wr0, semG0, semS0),
                (srcv1, dstv1, dstS1, xlr1, exr1, wr1, semG1, semS1)]

        def issue(ci, b):
            srcv, dstv, _, xlr, exr, _, semG, _ = bufs[b]
            base = wid * pw + ci * CHB
            pltpu.sync_copy(src_hbm.at[pl.ds(base, CHB)], srcv)
            pltpu.sync_copy(dst_hbm.at[pl.ds(base, CHB)], dstv)
            pltpu.async_copy(xl_hbm.at[srcv], xlr, semG)
            pltpu.async_copy(ex_hbm.at[pl.ds(base, CHB)], exr, semG)

        def drain_gather(b):
            srcv, _, _, xlr, exr, _, semG, _ = bufs[b]
            pltpu.make_async_copy(xl_hbm.at[srcv], xlr, semG).wait()
            pltpu.make_async_copy(ex_hbm.at[pl.ds(0, CHB)], exr, semG).wait()

        def drain_scatter(b):
            _, _, dstS, _, _, wr, _, semS = bufs[b]
            pltpu.make_async_copy(wr, out_sh.at[dstS], semS).wait()

        issue(0, 0)

        def two(i2, carry):
            for b in (0, 1):
                ci = i2 * 2 + b
                nb = 1 - b
                srcv, dstv, dstS, xlr, exr, wr, semG, semS = bufs[b]

                @pl.when(ci + 1 < nchunks)
                def _():
                    issue(ci + 1, nb)

                drain_gather(b)

                @pl.when(ci >= 2)
                def _():
                    drain_scatter(b)

                @plsc.parallel_loop(0, CHB, 1, unroll=4)
                def edge(e):
                    erow = jnp.full((16,), e, I32)
                    a16 = plsc.load_gather(exr, [erow, hsel])
                    for h in range(h_heads):
                        sl = pl.ds(h * hc, hc)
                        wr[e, sl] = xlr[e, sl] * a16[h]

                for k in range(CHB // 16):
                    ksl = pl.ds(k * 16, 16)
                    dstS[ksl] = dstv[ksl]
                pltpu.async_copy(wr, out_sh.at[dstS], semS, add=True)
            return carry

        lax.fori_loop(0, nchunks // 2, two, 0)
        drain_scatter(0)
        drain_scatter(1)
        plsc.subcore_barrier()

        def ofire(j, ocarry):
            sl = pl.ds(r0 + 8 * j, 8)
            pltpu.async_copy(out_sh.at[sl], out_hbm.at[c].at[sl], semZ)
            return ocarry

        def odrain(j, ocarry):
            sl0 = pl.ds(r0, 8)
            pltpu.make_async_copy(
                out_sh.at[sl0], out_hbm.at[c].at[sl0], semZ).wait()
            return ocarry

        lax.fori_loop(0, rpt // 8, ofire, 0)
        lax.fori_loop(0, rpt // 8, odrain, 0)

    vm = pltpu.VMEM
    return pl.kernel(
        body,
        out_type=jax.ShapeDtypeStruct((NC, n_out, h_heads * hc), F32),
        mesh=mesh,
        compiler_params=pltpu.CompilerParams(needs_layout_passes=False),
        scratch_types=[
            vm((CHB,), I32), vm((CHB,), I32), vm((CHB,), I32),
            vm((CHB, h_heads * hc), F32), vm((CHB, h_heads), F32),
            vm((CHB, h_heads * hc), F32),
            vm((CHB,), I32), vm((CHB,), I32), vm((CHB,), I32),
            vm((CHB, h_heads * hc), F32), vm((CHB, h_heads), F32),
            vm((CHB, h_heads * hc), F32),
            pltpu.VMEM_SHARED((n_out, h_heads * hc), F32),
            pltpu.SemaphoreType.DMA, pltpu.SemaphoreType.DMA,
            pltpu.SemaphoreType.DMA, pltpu.SemaphoreType.DMA,
            pltpu.SemaphoreType.DMA,
        ],
    )


def _tc_init_body(x_ref, wl_ref, bl_ref, wr_ref, br_ref, xl_ref, xr_ref):
    xb = x_ref[...]
    xl_ref[...] = jnp.dot(xb, wl_ref[...], preferred_element_type=F32) + bl_ref[...]
    xr_ref[...] = jnp.dot(xb, wr_ref[...], preferred_element_type=F32) + br_ref[...]


def _tc_em_body(ea_ref, we_ref, em_ref):
    em_ref[0] = jnp.dot(ea_ref[...], we_ref[0], preferred_element_type=F32)


def _den_widen(den_ref, nb, nheads, hc):
    dnb = den_ref[...] + 1e-16
    return jnp.concatenate(
        [jnp.broadcast_to(dnb[:, h:h + 1], (nb, hc)) for h in range(nheads)],
        axis=1)


def _tc_epi_body(nb, nheads, hc, p_ref, den_ref, cb_ref, lg_ref, lb_ref,
                 h_ref, wl_ref, bl_ref, wr_ref, br_ref, h_out, xl_out, xr_out):
    o = (p_ref[0] + p_ref[1]) / _den_widen(den_ref, nb, nheads, hc) + cb_ref[...]
    o = jnp.where(o > 0, o, jnp.exp(o) - 1.0)
    mu = jnp.mean(o, axis=-1, keepdims=True)
    var = jnp.mean((o - mu) * (o - mu), axis=-1, keepdims=True)
    o = (o - mu) / jnp.sqrt(var + 1e-5) * lg_ref[...] + lb_ref[...]
    hn = o + h_ref[...]
    h_out[...] = hn
    xl_out[...] = jnp.dot(hn, wl_ref[...], preferred_element_type=F32) + bl_ref[...]
    xr_out[...] = jnp.dot(hn, wr_ref[...], preferred_element_type=F32) + br_ref[...]


def _tc_final_body(nb, nheads, hc, p_ref, den_ref, cb_ref, lg_ref, lb_ref,
                   wp_ref, bp_ref, y_ref):
    o = (p_ref[0] + p_ref[1]) / _den_widen(den_ref, nb, nheads, hc)
    mh = o[:, 0:hc]
    for h in range(1, nheads):
        mh = mh + o[:, h * hc:(h + 1) * hc]
    mh = mh * (1.0 / nheads) + cb_ref[...]
    mh = jnp.where(mh > 0, mh, jnp.exp(mh) - 1.0)
    mu = jnp.mean(mh, axis=-1, keepdims=True)
    var = jnp.mean((mh - mu) * (mh - mu), axis=-1, keepdims=True)
    mh = (mh - mu) / jnp.sqrt(var + 1e-5) * lg_ref[...] + lb_ref[...]
    y_ref[...] = jnp.dot(mh, wp_ref[...], preferred_element_type=F32) + bp_ref[...]


def kernel(x, edge_index, edge_attr,
           Wl0, bl0, Wr0, br0, att0, We0, cb0, lg0, lb0,
           Wl1, bl1, Wr1, br1, att1, We1, cb1, lg1, lb1,
           Wl2, bl2, Wr2, br2, att2, We2, cb2, lg2, lb2,
           Wp, bp):
    n, d = x.shape
    e = edge_index.shape[1]
    hh, hc = att0.shape
    dhid = hh * hc
    ne = e + n
    ne_pad = ((ne + NW * CH - 1) // (NW * CH)) * (NW * CH)
    n_out = ((n + 1 + NS * 8 - 1) // (NS * 8)) * (NS * 8)
    nb = 1000
    ngrid = n // nb
    eb = 4096
    egrid = ne_pad // eb

    loop = jnp.arange(n, dtype=edge_index.dtype)
    pad = ne_pad - ne
    src = jnp.concatenate([edge_index[0], loop,
                           jnp.zeros((pad,), edge_index.dtype)])
    dst = jnp.concatenate([edge_index[1], loop,
                           jnp.full((pad,), n, edge_index.dtype)])
    fill = jnp.mean(edge_attr, axis=0, keepdims=True)
    ea = jnp.concatenate([edge_attr, jnp.tile(fill, (n, 1)),
                          jnp.zeros((pad, edge_attr.shape[1]), F32)])
    we_all = jnp.stack([We0, We1, We2])
    z128 = jnp.zeros((n_out, dhid), F32)

    xl, xr = pl.pallas_call(
        _tc_init_body,
        grid=(ngrid,),
        in_specs=[
            pl.BlockSpec((nb, d), lambda i: (i, 0)),
            pl.BlockSpec((d, dhid), lambda i: (0, 0)),
            pl.BlockSpec((1, dhid), lambda i: (0, 0)),
            pl.BlockSpec((d, dhid), lambda i: (0, 0)),
            pl.BlockSpec((1, dhid), lambda i: (0, 0)),
        ],
        out_specs=[
            pl.BlockSpec((nb, dhid), lambda i: (i, 0)),
            pl.BlockSpec((nb, dhid), lambda i: (i, 0)),
        ],
        out_shape=[
            jax.ShapeDtypeStruct((n, dhid), F32),
            jax.ShapeDtypeStruct((n, dhid), F32),
        ],
    )(x, Wl0, bl0.reshape(1, -1), Wr0, br0.reshape(1, -1))

    em = pl.pallas_call(
        _tc_em_body,
        grid=(3, egrid),
        in_specs=[
            pl.BlockSpec((eb, ea.shape[1]), lambda l, j: (j, 0)),
            pl.BlockSpec((1, ea.shape[1], dhid), lambda l, j: (l, 0, 0)),
        ],
        out_specs=pl.BlockSpec((1, eb, dhid), lambda l, j: (l, j, 0)),
        out_shape=jax.ShapeDtypeStruct((3, ne_pad, dhid), F32),
    )(ea, we_all)

    layer_params = [
        (cb0, lg0, lb0, Wl1, bl1, Wr1, br1, att0),
        (cb1, lg1, lb1, Wl2, bl2, Wr2, br2, att1),
        (cb2, lg2, lb2, None, None, None, None, att2),
    ]

    h = x
    y = None
    for l in range(3):
        cb, lg, lb, wl_n, bl_n, wr_n, br_n, att = layer_params[l]
        passA = _make_passA(l, ne_pad, n_out, hh, hc)
        ex = passA(src, dst, xl, xr, em, att)
        passD = _make_passD(ne_pad, n_out, hh)
        den = passD(dst, ex)
        dent = den.sum(axis=0).reshape(n_out, hh)
        passB2 = _make_passB2(ne_pad, n_out, hh, hc)
        outp = passB2(src, dst, xl, ex, z128)

        if l < 2:
            h, xl, xr = pl.pallas_call(
                functools.partial(_tc_epi_body, nb, hh, hc),
                grid=(ngrid,),
                in_specs=[
                    pl.BlockSpec((NC, nb, dhid), lambda i: (0, i, 0)),
                    pl.BlockSpec((nb, hh), lambda i: (i, 0)),
                    pl.BlockSpec((1, dhid), lambda i: (0, 0)),
                    pl.BlockSpec((1, dhid), lambda i: (0, 0)),
                    pl.BlockSpec((1, dhid), lambda i: (0, 0)),
                    pl.BlockSpec((nb, dhid), lambda i: (i, 0)),
                    pl.BlockSpec((d, dhid), lambda i: (0, 0)),
                    pl.BlockSpec((1, dhid), lambda i: (0, 0)),
                    pl.BlockSpec((d, dhid), lambda i: (0, 0)),
                    pl.BlockSpec((1, dhid), lambda i: (0, 0)),
                ],
                out_specs=[
                    pl.BlockSpec((nb, dhid), lambda i: (i, 0)),
                    pl.BlockSpec((nb, dhid), lambda i: (i, 0)),
                    pl.BlockSpec((nb, dhid), lambda i: (i, 0)),
                ],
                out_shape=[
                    jax.ShapeDtypeStruct((n, dhid), F32),
                    jax.ShapeDtypeStruct((n, dhid), F32),
                    jax.ShapeDtypeStruct((n, dhid), F32),
                ],
            )(outp, dent, cb.reshape(1, -1), lg.reshape(1, -1),
              lb.reshape(1, -1), h, wl_n, bl_n.reshape(1, -1), wr_n,
              br_n.reshape(1, -1))
        else:
            y = pl.pallas_call(
                functools.partial(_tc_final_body, nb, hh, hc),
                grid=(ngrid,),
                in_specs=[
                    pl.BlockSpec((NC, nb, dhid), lambda i: (0, i, 0)),
                    pl.BlockSpec((nb, hh), lambda i: (i, 0)),
                    pl.BlockSpec((1, hc), lambda i: (0, 0)),
                    pl.BlockSpec((1, hc), lambda i: (0, 0)),
                    pl.BlockSpec((1, hc), lambda i: (0, 0)),
                    pl.BlockSpec((hc, d), lambda i: (0, 0)),
                    pl.BlockSpec((1, d), lambda i: (0, 0)),
                ],
                out_specs=pl.BlockSpec((nb, d), lambda i: (i, 0)),
                out_shape=jax.ShapeDtypeStruct((n, d), F32),
            )(outp, dent, cb.reshape(1, -1), lg.reshape(1, -1),
              lb.reshape(1, -1), Wp, bp.reshape(1, -1))
    return y
